# baseline (speedup 1.0000x reference)
import functools
import math

import jax
import jax.numpy as jnp
from jax.experimental import pallas as pl
from jax.experimental.pallas import tpu as pltpu
from jax.experimental.pallas import tpu_sc as plsc

B, T, D, V, DM, DH = 1, 2048, 4, 8192, 1024, 4096
LN_EPS = 1e-05

_NSC = 32
_TT = 128
_HC = 1024
_QR = 1024
_QC = 2048


def _sc_gather(emb, ids):
    mesh = plsc.VectorSubcoreMesh(core_axis_name="c", subcore_axis_name="s")
    per_w = T // _NSC

    @functools.partial(
        pl.kernel,
        out_type=jax.ShapeDtypeStruct((T, DM), emb.dtype),
        mesh=mesh,
        scratch_types=[
            pltpu.VMEM((per_w,), jnp.int32),
            pltpu.VMEM((per_w, DM), jnp.float32),
            pltpu.SemaphoreType.DMA,
        ],
    )
    def gather_kernel(emb_hbm, ids_hbm, out_hbm, idx_v, rows_v, sem):
        wid = jax.lax.axis_index("s") * 2 + jax.lax.axis_index("c")
        base = wid * per_w
        pltpu.sync_copy(ids_hbm.at[pl.ds(base, per_w)], idx_v)
        pltpu.async_copy(emb_hbm.at[idx_v], rows_v, sem).wait()
        pltpu.sync_copy(rows_v, out_hbm.at[pl.ds(base, per_w)])

    return gather_kernel(emb, ids)


def _mlp_body(h_ref, e_ref, demb_ref, lng_ref, lnb_ref, w1_ref, b1_ref,
              w2_ref, b2_ref, wb_ref, bb_ref, s2_ref, beta_ref,
              w1_bf, w2_bf, stage, sem):
    R = _TT * D
    step = pl.program_id(0)

    @pl.when(step == 0)
    def _load_w1():
        for c in range(4):
            cp = pltpu.make_async_copy(
                w1_ref.at[:, pl.ds(c * 1024, 1024)],
                stage.at[pl.ds(0, DM), :], sem)
            cp.start()
            cp.wait()
            w1_bf[:, c * 1024:(c + 1) * 1024] = stage[0:DM, :].astype(jnp.bfloat16)
        pltpu.make_async_copy(w2_ref, stage, sem).start()

    x = h_ref[...] + e_ref[...]
    x4 = jnp.broadcast_to(x[:, None, :], (_TT, D, DM)).reshape(R, DM)
    d4 = jnp.broadcast_to(demb_ref[...][None, :, :], (_TT, D, DM)).reshape(R, DM)
    s = x4 + d4
    mu = jnp.mean(s, axis=1, keepdims=True)
    c = s - mu
    var = jnp.mean(c * c, axis=1, keepdims=True)
    sn = c * jax.lax.rsqrt(var + LN_EPS) * lng_ref[...] + lnb_ref[...]
    sn_bf = sn.astype(jnp.bfloat16)
    h_chunks = []
    for hc in range(DH // _HC):
        hpre = jnp.dot(sn_bf, w1_bf[:, hc * _HC:(hc + 1) * _HC],
                       preferred_element_type=jnp.float32
                       ) + b1_ref[:, hc * _HC:(hc + 1) * _HC]
        h = 0.5 * hpre * (1.0 + jax.lax.erf(hpre * (1.0 / math.sqrt(2.0))))
        h_chunks.append(h.astype(jnp.bfloat16))
    h_bf = jnp.concatenate(h_chunks, axis=1)

    @pl.when(step == 0)
    def _finish_w2():
        pltpu.make_async_copy(w2_ref, stage, sem).wait()
        w2_bf[...] = stage[...].astype(jnp.bfloat16)

    s2 = jnp.dot(h_bf, w2_bf[...],
                 preferred_element_type=jnp.float32) + b2_ref[...]
    s2_ref[...] = s2.astype(jnp.bfloat16)
    beta_ref[...] = (jnp.sum(s2 * wb_ref[...], axis=1) + bb_ref[0, 0]).reshape(1, _TT, D)


def _q_body(s2_ref, wq_ref, bq_ref, q_ref):
    qt = jnp.dot(s2_ref[...], wq_ref[...],
                 preferred_element_type=jnp.float32) + bq_ref[...]
    q_ref[...] = qt.reshape(1, _QR // D, D, _QC)


def kernel(H, input_ids, emb, depth_emb, ln_g, ln_b, W1, b1, W2, b2, Wq, bq, Wb, bb):
    f32 = jnp.float32
    bf16 = jnp.bfloat16

    H2 = H.reshape(T, DM)
    ids = input_ids.reshape(T).astype(jnp.int32)

    E = _sc_gather(emb, ids)

    R = _TT * D
    n_a = T // _TT
    s2_flat, beta = pl.pallas_call(
        _mlp_body,
        grid=(n_a,),
        in_specs=[
            pl.BlockSpec((_TT, DM), lambda i: (i, 0)),
            pl.BlockSpec((_TT, DM), lambda i: (i, 0)),
            pl.BlockSpec((D, DM), lambda i: (0, 0)),
            pl.BlockSpec((1, DM), lambda i: (0, 0)),
            pl.BlockSpec((1, DM), lambda i: (0, 0)),
            pl.BlockSpec(memory_space=pl.ANY),
            pl.BlockSpec((1, DH), lambda i: (0, 0)),
            pl.BlockSpec(memory_space=pl.ANY),
            pl.BlockSpec((1, DM), lambda i: (0, 0)),
            pl.BlockSpec((1, DM), lambda i: (0, 0)),
            pl.BlockSpec((1, 1), lambda i: (0, 0)),
        ],
        out_specs=[
            pl.BlockSpec((R, DM), lambda i: (i, 0)),
            pl.BlockSpec((1, _TT, D), lambda i: (0, i, 0)),
        ],
        out_shape=[
            jax.ShapeDtypeStruct((T * D, DM), bf16),
            jax.ShapeDtypeStruct((B, T, D), f32),
        ],
        scratch_shapes=[
            pltpu.VMEM((DM, DH), bf16),
            pltpu.VMEM((DH, DM), bf16),
            pltpu.VMEM((DH, DM), f32),
            pltpu.SemaphoreType.DMA,
        ],
    )(
        H2, E, depth_emb,
        ln_g.reshape(1, DM), ln_b.reshape(1, DM),
        W1, b1.reshape(1, DH),
        W2, b2.reshape(1, DM),
        Wb.reshape(1, DM), bb.reshape(1, 1),
    )

    n_qr = (T * D) // _QR
    n_qc = V // _QC
    q = pl.pallas_call(
        _q_body,
        grid=(n_qc, n_qr),
        in_specs=[
            pl.BlockSpec((_QR, DM), lambda j, i: (i, 0)),
            pl.BlockSpec((DM, _QC), lambda j, i: (0, j)),
            pl.BlockSpec((1, _QC), lambda j, i: (0, j)),
        ],
        out_specs=pl.BlockSpec((1, _QR // D, D, _QC),
                               lambda j, i: (0, i, 0, j)),
        out_shape=jax.ShapeDtypeStruct((B, T, D, V), f32),
    )(s2_flat, Wq.astype(bf16), bq.reshape(1, V))

    return (q, beta)


# speedup vs baseline: 1.0223x; 1.0223x over previous
import functools
import math

import jax
import jax.numpy as jnp
from jax.experimental import pallas as pl
from jax.experimental.pallas import tpu as pltpu
from jax.experimental.pallas import tpu_sc as plsc

B, T, D, V, DM, DH = 1, 2048, 4, 8192, 1024, 4096
LN_EPS = 1e-05

_NSC = 32
_TT = 128
_QR = 1024
_QC = 2048


def _sc_gather(emb, ids):
    mesh = plsc.VectorSubcoreMesh(core_axis_name="c", subcore_axis_name="s")
    per_w = T // _NSC

    @functools.partial(
        pl.kernel,
        out_type=jax.ShapeDtypeStruct((T, DM), emb.dtype),
        mesh=mesh,
        scratch_types=[
            pltpu.VMEM((per_w,), jnp.int32),
            pltpu.VMEM((per_w, DM), jnp.float32),
            pltpu.SemaphoreType.DMA,
        ],
    )
    def gather_kernel(emb_hbm, ids_hbm, out_hbm, idx_v, rows_v, sem):
        wid = jax.lax.axis_index("s") * 2 + jax.lax.axis_index("c")
        base = wid * per_w
        pltpu.sync_copy(ids_hbm.at[pl.ds(base, per_w)], idx_v)
        pltpu.async_copy(emb_hbm.at[idx_v], rows_v, sem).wait()
        pltpu.sync_copy(rows_v, out_hbm.at[pl.ds(base, per_w)])

    return gather_kernel(emb, ids)


def _mlp_body(h_ref, e_ref, demb_ref, lng_ref, lnb_ref, w1_ref, b1_ref,
              w2_ref, b2_ref, wb_ref, bb_ref, s2_ref, beta_ref):
    R = _TT * D
    x = h_ref[...] + e_ref[...]
    x4 = jnp.broadcast_to(x[:, None, :], (_TT, D, DM)).reshape(R, DM)
    d4 = jnp.broadcast_to(demb_ref[...][None, :, :], (_TT, D, DM)).reshape(R, DM)
    s = x4 + d4
    mu = jnp.mean(s, axis=1, keepdims=True)
    c = s - mu
    var = jnp.mean(c * c, axis=1, keepdims=True)
    sn = c * jax.lax.rsqrt(var + LN_EPS) * lng_ref[...] + lnb_ref[...]
    hpre = jnp.dot(sn.astype(jnp.bfloat16), w1_ref[...],
                   preferred_element_type=jnp.float32) + b1_ref[...]
    h = 0.5 * hpre * (1.0 + jax.lax.erf(hpre * (1.0 / math.sqrt(2.0))))
    s2 = jnp.dot(h.astype(jnp.bfloat16), w2_ref[...],
                 preferred_element_type=jnp.float32) + b2_ref[...]
    s2_ref[...] = s2.astype(jnp.bfloat16)
    beta_ref[...] = (jnp.sum(s2 * wb_ref[...], axis=1) + bb_ref[0, 0]).reshape(1, _TT, D)


def _q_body(s2_ref, wq_ref, bq_ref, q_ref):
    qt = jnp.dot(s2_ref[...], wq_ref[...],
                 preferred_element_type=jnp.float32) + bq_ref[...]
    q_ref[...] = qt.reshape(1, _QR // D, D, _QC)


def kernel(H, input_ids, emb, depth_emb, ln_g, ln_b, W1, b1, W2, b2, Wq, bq, Wb, bb):
    f32 = jnp.float32
    bf16 = jnp.bfloat16

    H2 = H.reshape(T, DM)
    ids = input_ids.reshape(T).astype(jnp.int32)

    E = _sc_gather(emb, ids)

    R = _TT * D
    n_a = T // _TT
    s2_flat, beta = pl.pallas_call(
        _mlp_body,
        grid=(n_a,),
        in_specs=[
            pl.BlockSpec((_TT, DM), lambda i: (i, 0)),
            pl.BlockSpec((_TT, DM), lambda i: (i, 0)),
            pl.BlockSpec((D, DM), lambda i: (0, 0)),
            pl.BlockSpec((1, DM), lambda i: (0, 0)),
            pl.BlockSpec((1, DM), lambda i: (0, 0)),
            pl.BlockSpec((DM, DH), lambda i: (0, 0)),
            pl.BlockSpec((1, DH), lambda i: (0, 0)),
            pl.BlockSpec((DH, DM), lambda i: (0, 0)),
            pl.BlockSpec((1, DM), lambda i: (0, 0)),
            pl.BlockSpec((1, DM), lambda i: (0, 0)),
            pl.BlockSpec((1, 1), lambda i: (0, 0)),
        ],
        out_specs=[
            pl.BlockSpec((R, DM), lambda i: (i, 0)),
            pl.BlockSpec((1, _TT, D), lambda i: (0, i, 0)),
        ],
        out_shape=[
            jax.ShapeDtypeStruct((T * D, DM), bf16),
            jax.ShapeDtypeStruct((B, T, D), f32),
        ],
    )(
        H2, E, depth_emb,
        ln_g.reshape(1, DM), ln_b.reshape(1, DM),
        W1.astype(bf16), b1.reshape(1, DH),
        W2.astype(bf16), b2.reshape(1, DM),
        Wb.reshape(1, DM), bb.reshape(1, 1),
    )

    n_qr = (T * D) // _QR
    n_qc = V // _QC
    q = pl.pallas_call(
        _q_body,
        grid=(n_qc, n_qr),
        in_specs=[
            pl.BlockSpec((_QR, DM), lambda j, i: (i, 0)),
            pl.BlockSpec((DM, _QC), lambda j, i: (0, j)),
            pl.BlockSpec((1, _QC), lambda j, i: (0, j)),
        ],
        out_specs=pl.BlockSpec((1, _QR // D, D, _QC),
                               lambda j, i: (0, i, 0, j)),
        out_shape=jax.ShapeDtypeStruct((B, T, D, V), f32),
    )(s2_flat, Wq.astype(bf16), bq.reshape(1, V))

    return (q, beta)
